# Initial kernel scaffold; baseline (speedup 1.0000x reference)
import jax
import jax.numpy as jnp
from jax.experimental import pallas as pl


def kernel(x, edge_index, first_edge, candidate_edges, t, gin0_W1, gin0_b1, gin0_W2, gin0_b2, gin1_W1, gin1_b1, gin1_W2, gin1_b2, gin2_W1, gin2_b1, gin2_W2, gin2_b2, ep_W1, ep_b1, ep_W2, ep_b2, te_W1, te_b1, te_W2, te_b2):
    raise NotImplementedError("write your pallas kernel here")


# speedup vs baseline: 4.5448x; 4.5448x over previous
import functools

import jax
import jax.numpy as jnp
from jax import lax
from jax.experimental import pallas as pl
from jax.experimental.pallas import tpu as pltpu
from jax.experimental.pallas import tpu_sc as plsc

_NC = 2
_NS = 16
_NW = _NC * _NS


def _sc_aggregate(x, src, dst):
    N, D = x.shape
    E = src.shape[0]
    EW = E // _NW
    CH = 80
    NCH = EW // CH
    RPT = (N // _NS) // 8 * 8
    RPT_LAST = N - (_NS - 1) * RPT
    mesh = plsc.VectorSubcoreMesh(core_axis_name="c", subcore_axis_name="s")

    @functools.partial(
        pl.kernel,
        out_type=jax.ShapeDtypeStruct((_NC, N, D), jnp.float32),
        mesh=mesh,
        scratch_types=[
            pltpu.VMEM((CH,), jnp.int32),
            pltpu.VMEM((CH,), jnp.int32),
            pltpu.VMEM((CH, D), jnp.float32),
            pltpu.VMEM_SHARED((N, D), jnp.float32),
            pltpu.SemaphoreType.DMA,
        ],
    )
    def agg_kernel(x_hbm, src_hbm, dst_hbm, out_hbm, src_v, dst_v, rows_v,
                   acc_sh, sem):
        cid = lax.axis_index("c")
        sid = lax.axis_index("s")
        wid = cid * _NS + sid

        zeros16 = jnp.zeros((16,), jnp.float32)

        def zrow(i, carry):
            for j in range(D // 16):
                rows_v[i, pl.ds(j * 16, 16)] = zeros16
            return carry

        lax.fori_loop(0, CH, zrow, 0)

        @pl.when(sid < _NS - 1)
        def _zero_main():
            for k in range(RPT // CH):
                pltpu.sync_copy(rows_v,
                                acc_sh.at[pl.ds(sid * RPT + k * CH, CH)])
            rem = RPT % CH
            if rem:
                pltpu.sync_copy(
                    rows_v.at[pl.ds(0, rem)],
                    acc_sh.at[pl.ds(sid * RPT + (RPT // CH) * CH, rem)])

        @pl.when(sid == _NS - 1)
        def _zero_last():
            base = (_NS - 1) * RPT
            for k in range(RPT_LAST // CH):
                pltpu.sync_copy(rows_v, acc_sh.at[pl.ds(base + k * CH, CH)])
            rem = RPT_LAST % CH
            if rem:
                pltpu.sync_copy(
                    rows_v.at[pl.ds(0, rem)],
                    acc_sh.at[pl.ds(base + (RPT_LAST // CH) * CH, rem)])

        plsc.subcore_barrier()

        def chunk(c, carry):
            off = wid * EW + c * CH
            pltpu.sync_copy(src_hbm.at[pl.ds(off, CH)], src_v)
            pltpu.sync_copy(dst_hbm.at[pl.ds(off, CH)], dst_v)
            pltpu.async_copy(x_hbm.at[src_v], rows_v, sem).wait()
            pltpu.sync_copy(rows_v, acc_sh.at[dst_v], add=True)
            return carry

        lax.fori_loop(0, NCH, chunk, 0)
        plsc.subcore_barrier()

        @pl.when(sid < _NS - 1)
        def _out_main():
            pltpu.sync_copy(acc_sh.at[pl.ds(sid * RPT, RPT)],
                            out_hbm.at[cid, pl.ds(sid * RPT, RPT)])

        @pl.when(sid == _NS - 1)
        def _out_last():
            base = (_NS - 1) * RPT
            pltpu.sync_copy(acc_sh.at[pl.ds(base, RPT_LAST)],
                            out_hbm.at[cid, pl.ds(base, RPT_LAST)])

    return agg_kernel(x, src, dst)


def _tc_mlp(x, agg, W1, b1, W2, b2):
    N, D = x.shape
    H = W1.shape[1]
    BR = 2000

    def body(x_ref, a_ref, w1_ref, b1_ref, w2_ref, b2_ref, o_ref):
        s = x_ref[...] + a_ref[0] + a_ref[1]
        h = jnp.dot(s, w1_ref[...], preferred_element_type=jnp.float32)
        h = jnp.maximum(h + b1_ref[...], 0.0)
        o_ref[...] = (jnp.dot(h, w2_ref[...],
                              preferred_element_type=jnp.float32) + b2_ref[...])

    return pl.pallas_call(
        body,
        grid=(N // BR,),
        in_specs=[
            pl.BlockSpec((BR, D), lambda i: (i, 0)),
            pl.BlockSpec((_NC, BR, D), lambda i: (0, i, 0)),
            pl.BlockSpec((D, H), lambda i: (0, 0)),
            pl.BlockSpec((1, H), lambda i: (0, 0)),
            pl.BlockSpec((H, H), lambda i: (0, 0)),
            pl.BlockSpec((1, H), lambda i: (0, 0)),
        ],
        out_specs=pl.BlockSpec((BR, H), lambda i: (i, 0)),
        out_shape=jax.ShapeDtypeStruct((N, H), jnp.float32),
    )(x, agg, W1, b1, W2, b2)


def _sc_gather(x, uidx, vidx):
    N, D = x.shape
    B = uidx.shape[0]
    BW = B // _NW
    mesh = plsc.VectorSubcoreMesh(core_axis_name="c", subcore_axis_name="s")

    @functools.partial(
        pl.kernel,
        out_type=(jax.ShapeDtypeStruct((B, D), jnp.float32),
                  jax.ShapeDtypeStruct((B, D), jnp.float32)),
        mesh=mesh,
        scratch_types=[
            pltpu.VMEM((BW,), jnp.int32),
            pltpu.VMEM((BW, D), jnp.float32),
            pltpu.SemaphoreType.DMA,
        ],
    )
    def gather_kernel(x_hbm, u_hbm, v_hbm, ou_hbm, ov_hbm, idx_v, rows_v, sem):
        cid = lax.axis_index("c")
        sid = lax.axis_index("s")
        base = (cid * _NS + sid) * BW
        pltpu.sync_copy(u_hbm.at[pl.ds(base, BW)], idx_v)
        pltpu.async_copy(x_hbm.at[idx_v], rows_v, sem).wait()
        pltpu.sync_copy(rows_v, ou_hbm.at[pl.ds(base, BW)])
        pltpu.sync_copy(v_hbm.at[pl.ds(base, BW)], idx_v)
        pltpu.async_copy(x_hbm.at[idx_v], rows_v, sem).wait()
        pltpu.sync_copy(rows_v, ov_hbm.at[pl.ds(base, BW)])

    return gather_kernel(x, uidx, vidx)


def _tc_score(xu, xv, fu, fv, tb, ep_W1, ep_b1, ep_W2, ep_b2,
              te_W1, te_b1, te_W2, te_b2):
    Cn, H = xu.shape

    def body(xu_ref, xv_ref, fu_ref, fv_ref, tb_ref, w1_ref, b1_ref, w2_ref,
             b2_ref, tw1_ref, tb1_ref, tw2_ref, tb2_ref, o_ref):
        w1 = w1_ref[...]
        s = xu_ref[...] + xv_ref[...]
        d = jnp.abs(xu_ref[...] - xv_ref[...])
        ffs = fu_ref[...] + fv_ref[...]
        ffd = jnp.abs(fu_ref[...] - fv_ref[...])
        te = jnp.maximum(tb_ref[...] * tw1_ref[...] + tb1_ref[...], 0.0)
        temb = (jnp.dot(te, tw2_ref[...], preferred_element_type=jnp.float32)
                + tb2_ref[...])
        cvec = (jnp.dot(ffs, w1[0:H, :], preferred_element_type=jnp.float32)
                + jnp.dot(ffd, w1[H:2 * H, :],
                          preferred_element_type=jnp.float32)
                + jnp.dot(temb, w1[4 * H:5 * H, :],
                          preferred_element_type=jnp.float32)
                + b1_ref[...])
        pre = (jnp.dot(s, w1[2 * H:3 * H, :],
                       preferred_element_type=jnp.float32)
               + jnp.dot(d, w1[3 * H:4 * H, :],
                         preferred_element_type=jnp.float32)
               + cvec)
        h = jnp.maximum(pre, 0.0)
        o_ref[...] = (jnp.dot(h, w2_ref[...],
                              preferred_element_type=jnp.float32) + b2_ref[...])

    return pl.pallas_call(
        body,
        out_shape=jax.ShapeDtypeStruct((Cn, 1), jnp.float32),
    )(xu, xv, fu, fv, tb, ep_W1, ep_b1, ep_W2, ep_b2,
      te_W1, te_b1, te_W2, te_b2)


def kernel(x, edge_index, first_edge, candidate_edges, t,
           gin0_W1, gin0_b1, gin0_W2, gin0_b2,
           gin1_W1, gin1_b1, gin1_W2, gin1_b2,
           gin2_W1, gin2_b1, gin2_W2, gin2_b2,
           ep_W1, ep_b1, ep_W2, ep_b2,
           te_W1, te_b1, te_W2, te_b2):
    N, D = x.shape
    H = gin0_W1.shape[1]
    src = edge_index[0]
    dst = edge_index[1]

    layers = ((gin0_W1, gin0_b1, gin0_W2, gin0_b2),
              (gin1_W1, gin1_b1, gin1_W2, gin1_b2),
              (gin2_W1, gin2_b1, gin2_W2, gin2_b2))
    for W1, b1, W2, b2 in layers:
        agg = _sc_aggregate(x, src, dst)
        x = _tc_mlp(x, agg, W1, b1.reshape(1, H), W2, b2.reshape(1, H))

    Cn = candidate_edges.shape[0]
    B = ((Cn + 1 + 8 * _NW - 1) // (8 * _NW)) * (8 * _NW)
    pad = jnp.arange(B - Cn - 1, dtype=jnp.int32) % N
    uidx = jnp.concatenate([candidate_edges[:, 0].astype(jnp.int32),
                            first_edge[0:1].astype(jnp.int32), pad])
    vidx = jnp.concatenate([candidate_edges[:, 1].astype(jnp.int32),
                            first_edge[1:2].astype(jnp.int32), pad])
    xu_all, xv_all = _sc_gather(x, uidx, vidx)
    xu = xu_all[:Cn]
    xv = xv_all[:Cn]
    fu = xu_all[Cn:Cn + 1]
    fv = xv_all[Cn:Cn + 1]

    tb = jnp.full((1, H), t, dtype=jnp.float32)
    scores = _tc_score(xu, xv, fu, fv, tb,
                       ep_W1, ep_b1.reshape(1, H), ep_W2,
                       ep_b2.reshape(1, 1),
                       te_W1, te_b1.reshape(1, H), te_W2,
                       te_b2.reshape(1, H))
    return scores.reshape(-1)


# speedup vs baseline: 7.3968x; 1.6275x over previous
import functools

import jax
import jax.numpy as jnp
from jax import lax
from jax.experimental import pallas as pl
from jax.experimental.pallas import tpu as pltpu
from jax.experimental.pallas import tpu_sc as plsc

_NC = 2
_NS = 16
_NW = _NC * _NS


def _sc_aggregate(x, src, dst):
    N, D = x.shape
    E = src.shape[0]
    EW = E // _NW
    CH = 80
    NCH = EW // CH
    RPT = (N // _NS) // 8 * 8
    RPT_LAST = N - (_NS - 1) * RPT
    mesh = plsc.VectorSubcoreMesh(core_axis_name="c", subcore_axis_name="s")

    @functools.partial(
        pl.kernel,
        out_type=jax.ShapeDtypeStruct((_NC, N, D), jnp.float32),
        mesh=mesh,
        scratch_types=[
            pltpu.VMEM((CH,), jnp.int32),
            pltpu.VMEM((CH,), jnp.int32),
            pltpu.VMEM((CH,), jnp.int32),
            pltpu.VMEM((CH,), jnp.int32),
            pltpu.VMEM((CH, D), jnp.float32),
            pltpu.VMEM((CH, D), jnp.float32),
            pltpu.VMEM_SHARED((N, D), jnp.float32),
            pltpu.SemaphoreType.DMA,
            pltpu.SemaphoreType.DMA,
            pltpu.SemaphoreType.DMA,
            pltpu.SemaphoreType.DMA,
            pltpu.SemaphoreType.DMA,
            pltpu.SemaphoreType.DMA,
        ],
    )
    def agg_kernel(x_hbm, src_hbm, dst_hbm, out_hbm, sidx_a, didx_a, sidx_b,
                   didx_b, rows_a, rows_b, acc_sh, isa, isb, gsa, gsb, ssa,
                   ssb):
        cid = lax.axis_index("c")
        sid = lax.axis_index("s")
        wid = cid * _NS + sid
        rows_v = rows_a
        ebase = wid * EW

        def load_idx(c, sidx, didx, sem):
            pltpu.async_copy(src_hbm.at[pl.ds(ebase + c * CH, CH)], sidx, sem)
            pltpu.async_copy(dst_hbm.at[pl.ds(ebase + c * CH, CH)], didx, sem)

        def wait_idx(sidx, didx, sem):
            pltpu.make_async_copy(src_hbm.at[pl.ds(0, CH)], sidx, sem).wait()
            pltpu.make_async_copy(dst_hbm.at[pl.ds(0, CH)], didx, sem).wait()

        load_idx(0, sidx_a, didx_a, isa)
        load_idx(1, sidx_b, didx_b, isb)

        zeros16 = jnp.zeros((16,), jnp.float32)

        def zrow(i, carry):
            for j in range(D // 16):
                rows_v[i, pl.ds(j * 16, 16)] = zeros16
            return carry

        lax.fori_loop(0, CH, zrow, 0)

        @pl.when(sid < _NS - 1)
        def _zero_main():
            for k in range(RPT // CH):
                pltpu.sync_copy(rows_v,
                                acc_sh.at[pl.ds(sid * RPT + k * CH, CH)])
            rem = RPT % CH
            if rem:
                pltpu.sync_copy(
                    rows_v.at[pl.ds(0, rem)],
                    acc_sh.at[pl.ds(sid * RPT + (RPT // CH) * CH, rem)])

        @pl.when(sid == _NS - 1)
        def _zero_last():
            base = (_NS - 1) * RPT
            for k in range(RPT_LAST // CH):
                pltpu.sync_copy(rows_v, acc_sh.at[pl.ds(base + k * CH, CH)])
            rem = RPT_LAST % CH
            if rem:
                pltpu.sync_copy(
                    rows_v.at[pl.ds(0, rem)],
                    acc_sh.at[pl.ds(base + (RPT_LAST // CH) * CH, rem)])

        plsc.subcore_barrier()

        def pair(i, carry):
            c = 2 * i
            wait_idx(sidx_a, didx_a, isa)
            ga = pltpu.async_copy(x_hbm.at[sidx_a], rows_a, gsa)
            wait_idx(sidx_b, didx_b, isb)
            ga.wait()
            gb = pltpu.async_copy(x_hbm.at[sidx_b], rows_b, gsb)
            sa = pltpu.async_copy(rows_a, acc_sh.at[didx_a], ssa, add=True)
            gb.wait()
            sa.wait()
            load_idx(c + 2, sidx_a, didx_a, isa)
            sb = pltpu.async_copy(rows_b, acc_sh.at[didx_b], ssb, add=True)
            sb.wait()

            @pl.when(c + 3 < NCH)
            def _():
                load_idx(c + 3, sidx_b, didx_b, isb)

            return carry

        lax.fori_loop(0, NCH // 2, pair, 0)
        wait_idx(sidx_a, didx_a, isa)
        pltpu.async_copy(x_hbm.at[sidx_a], rows_a, gsa).wait()
        pltpu.sync_copy(rows_a, acc_sh.at[didx_a], add=True)
        plsc.subcore_barrier()

        @pl.when(sid < _NS - 1)
        def _out_main():
            pltpu.sync_copy(acc_sh.at[pl.ds(sid * RPT, RPT)],
                            out_hbm.at[cid, pl.ds(sid * RPT, RPT)])

        @pl.when(sid == _NS - 1)
        def _out_last():
            base = (_NS - 1) * RPT
            pltpu.sync_copy(acc_sh.at[pl.ds(base, RPT_LAST)],
                            out_hbm.at[cid, pl.ds(base, RPT_LAST)])

    return agg_kernel(x, src, dst)


def _tc_mlp(x, agg, W1, b1, W2, b2):
    N, D = x.shape
    H = W1.shape[1]
    BR = 2000

    def body(x_ref, a_ref, w1_ref, b1_ref, w2_ref, b2_ref, o_ref):
        s = x_ref[...] + a_ref[0] + a_ref[1]
        h = jnp.dot(s, w1_ref[...], preferred_element_type=jnp.float32)
        h = jnp.maximum(h + b1_ref[...], 0.0)
        o_ref[...] = (jnp.dot(h, w2_ref[...],
                              preferred_element_type=jnp.float32) + b2_ref[...])

    return pl.pallas_call(
        body,
        grid=(N // BR,),
        in_specs=[
            pl.BlockSpec((BR, D), lambda i: (i, 0)),
            pl.BlockSpec((_NC, BR, D), lambda i: (0, i, 0)),
            pl.BlockSpec((D, H), lambda i: (0, 0)),
            pl.BlockSpec((1, H), lambda i: (0, 0)),
            pl.BlockSpec((H, H), lambda i: (0, 0)),
            pl.BlockSpec((1, H), lambda i: (0, 0)),
        ],
        out_specs=pl.BlockSpec((BR, H), lambda i: (i, 0)),
        out_shape=jax.ShapeDtypeStruct((N, H), jnp.float32),
    )(x, agg, W1, b1, W2, b2)


def _sc_gather(x, uidx, vidx):
    N, D = x.shape
    B = uidx.shape[0]
    BW = B // _NW
    mesh = plsc.VectorSubcoreMesh(core_axis_name="c", subcore_axis_name="s")

    @functools.partial(
        pl.kernel,
        out_type=(jax.ShapeDtypeStruct((B, D), jnp.float32),
                  jax.ShapeDtypeStruct((B, D), jnp.float32)),
        mesh=mesh,
        scratch_types=[
            pltpu.VMEM((BW,), jnp.int32),
            pltpu.VMEM((BW, D), jnp.float32),
            pltpu.SemaphoreType.DMA,
        ],
    )
    def gather_kernel(x_hbm, u_hbm, v_hbm, ou_hbm, ov_hbm, idx_v, rows_v, sem):
        cid = lax.axis_index("c")
        sid = lax.axis_index("s")
        base = (cid * _NS + sid) * BW
        pltpu.sync_copy(u_hbm.at[pl.ds(base, BW)], idx_v)
        pltpu.async_copy(x_hbm.at[idx_v], rows_v, sem).wait()
        pltpu.sync_copy(rows_v, ou_hbm.at[pl.ds(base, BW)])
        pltpu.sync_copy(v_hbm.at[pl.ds(base, BW)], idx_v)
        pltpu.async_copy(x_hbm.at[idx_v], rows_v, sem).wait()
        pltpu.sync_copy(rows_v, ov_hbm.at[pl.ds(base, BW)])

    return gather_kernel(x, uidx, vidx)


def _tc_score(xu, xv, fu, fv, tb, ep_W1, ep_b1, ep_W2, ep_b2,
              te_W1, te_b1, te_W2, te_b2):
    Cn, H = xu.shape

    def body(xu_ref, xv_ref, fu_ref, fv_ref, tb_ref, w1_ref, b1_ref, w2_ref,
             b2_ref, tw1_ref, tb1_ref, tw2_ref, tb2_ref, o_ref):
        w1 = w1_ref[...]
        s = xu_ref[...] + xv_ref[...]
        d = jnp.abs(xu_ref[...] - xv_ref[...])
        ffs = fu_ref[...] + fv_ref[...]
        ffd = jnp.abs(fu_ref[...] - fv_ref[...])
        te = jnp.maximum(tb_ref[...] * tw1_ref[...] + tb1_ref[...], 0.0)
        temb = (jnp.dot(te, tw2_ref[...], preferred_element_type=jnp.float32)
                + tb2_ref[...])
        cvec = (jnp.dot(ffs, w1[0:H, :], preferred_element_type=jnp.float32)
                + jnp.dot(ffd, w1[H:2 * H, :],
                          preferred_element_type=jnp.float32)
                + jnp.dot(temb, w1[4 * H:5 * H, :],
                          preferred_element_type=jnp.float32)
                + b1_ref[...])
        pre = (jnp.dot(s, w1[2 * H:3 * H, :],
                       preferred_element_type=jnp.float32)
               + jnp.dot(d, w1[3 * H:4 * H, :],
                         preferred_element_type=jnp.float32)
               + cvec)
        h = jnp.maximum(pre, 0.0)
        o_ref[...] = (jnp.dot(h, w2_ref[...],
                              preferred_element_type=jnp.float32) + b2_ref[...])

    return pl.pallas_call(
        body,
        out_shape=jax.ShapeDtypeStruct((Cn, 1), jnp.float32),
    )(xu, xv, fu, fv, tb, ep_W1, ep_b1, ep_W2, ep_b2,
      te_W1, te_b1, te_W2, te_b2)


def kernel(x, edge_index, first_edge, candidate_edges, t,
           gin0_W1, gin0_b1, gin0_W2, gin0_b2,
           gin1_W1, gin1_b1, gin1_W2, gin1_b2,
           gin2_W1, gin2_b1, gin2_W2, gin2_b2,
           ep_W1, ep_b1, ep_W2, ep_b2,
           te_W1, te_b1, te_W2, te_b2):
    N, D = x.shape
    H = gin0_W1.shape[1]
    src = edge_index[0]
    dst = edge_index[1]

    layers = ((gin0_W1, gin0_b1, gin0_W2, gin0_b2),
              (gin1_W1, gin1_b1, gin1_W2, gin1_b2),
              (gin2_W1, gin2_b1, gin2_W2, gin2_b2))
    for W1, b1, W2, b2 in layers:
        agg = _sc_aggregate(x, src, dst)
        x = _tc_mlp(x, agg, W1, b1.reshape(1, H), W2, b2.reshape(1, H))

    Cn = candidate_edges.shape[0]
    B = ((Cn + 1 + 8 * _NW - 1) // (8 * _NW)) * (8 * _NW)
    pad = jnp.arange(B - Cn - 1, dtype=jnp.int32) % N
    uidx = jnp.concatenate([candidate_edges[:, 0].astype(jnp.int32),
                            first_edge[0:1].astype(jnp.int32), pad])
    vidx = jnp.concatenate([candidate_edges[:, 1].astype(jnp.int32),
                            first_edge[1:2].astype(jnp.int32), pad])
    xu_all, xv_all = _sc_gather(x, uidx, vidx)
    xu = xu_all[:Cn]
    xv = xv_all[:Cn]
    fu = xu_all[Cn:Cn + 1]
    fv = xv_all[Cn:Cn + 1]

    tb = jnp.full((1, H), t, dtype=jnp.float32)
    scores = _tc_score(xu, xv, fu, fv, tb,
                       ep_W1, ep_b1.reshape(1, H), ep_W2,
                       ep_b2.reshape(1, 1),
                       te_W1, te_b1.reshape(1, H), te_W2,
                       te_b2.reshape(1, H))
    return scores.reshape(-1)


# speedup vs baseline: 8.5040x; 1.1497x over previous
import functools

import jax
import jax.numpy as jnp
from jax import lax
from jax.experimental import pallas as pl
from jax.experimental.pallas import tpu as pltpu
from jax.experimental.pallas import tpu_sc as plsc

_NC = 2
_NS = 16
_NW = _NC * _NS


def _sc_aggregate(x, src, dst):
    N, D = x.shape
    E = src.shape[0]
    EW = E // _NW
    CH = 80
    NCH = EW // CH
    RPT = (N // _NS) // 8 * 8
    RPT_LAST = N - (_NS - 1) * RPT
    mesh = plsc.VectorSubcoreMesh(core_axis_name="c", subcore_axis_name="s")

    @functools.partial(
        pl.kernel,
        out_type=jax.ShapeDtypeStruct((_NC, N, D), jnp.float32),
        mesh=mesh,
        scratch_types=[
            pltpu.VMEM((CH,), jnp.int32),
            pltpu.VMEM((CH,), jnp.int32),
            pltpu.VMEM((CH,), jnp.int32),
            pltpu.VMEM((CH,), jnp.int32),
            pltpu.VMEM((CH, D), jnp.float32),
            pltpu.VMEM((CH, D), jnp.float32),
            pltpu.VMEM_SHARED((N, D), jnp.float32),
            pltpu.SemaphoreType.DMA,
            pltpu.SemaphoreType.DMA,
            pltpu.SemaphoreType.DMA,
            pltpu.SemaphoreType.DMA,
            pltpu.SemaphoreType.DMA,
            pltpu.SemaphoreType.DMA,
            pltpu.SemaphoreType.DMA,
            pltpu.SemaphoreType.DMA,
        ],
    )
    def agg_kernel(x_hbm, src_hbm, dst_hbm, out_hbm, sidx_a, didx_a, sidx_b,
                   didx_b, rows_a, rows_b, acc_sh, isas, isad, isbs, isbd,
                   gsa, gsb, ssa, ssb):
        cid = lax.axis_index("c")
        sid = lax.axis_index("s")
        wid = cid * _NS + sid
        rows_v = rows_a
        ebase = wid * EW

        def load_sidx(c, sidx, sem):
            pltpu.async_copy(src_hbm.at[pl.ds(ebase + c * CH, CH)], sidx, sem)

        def load_didx(c, didx, sem):
            pltpu.async_copy(dst_hbm.at[pl.ds(ebase + c * CH, CH)], didx, sem)

        def wait_i(buf, sem):
            pltpu.make_async_copy(src_hbm.at[pl.ds(0, CH)], buf, sem).wait()

        def wait_rows(buf, sem):
            pltpu.make_async_copy(x_hbm.at[pl.ds(0, CH)], buf, sem).wait()

        load_sidx(0, sidx_a, isas)
        load_didx(0, didx_a, isad)
        load_sidx(1, sidx_b, isbs)
        load_didx(1, didx_b, isbd)

        zeros16 = jnp.zeros((16,), jnp.float32)

        def zrow(i, carry):
            for j in range(D // 16):
                rows_v[i, pl.ds(j * 16, 16)] = zeros16
            return carry

        lax.fori_loop(0, CH, zrow, 0)

        @pl.when(sid < _NS - 1)
        def _zero_main():
            for k in range(RPT // CH):
                pltpu.sync_copy(rows_v,
                                acc_sh.at[pl.ds(sid * RPT + k * CH, CH)])
            rem = RPT % CH
            if rem:
                pltpu.sync_copy(
                    rows_v.at[pl.ds(0, rem)],
                    acc_sh.at[pl.ds(sid * RPT + (RPT // CH) * CH, rem)])

        @pl.when(sid == _NS - 1)
        def _zero_last():
            base = (_NS - 1) * RPT
            for k in range(RPT_LAST // CH):
                pltpu.sync_copy(rows_v, acc_sh.at[pl.ds(base + k * CH, CH)])
            rem = RPT_LAST % CH
            if rem:
                pltpu.sync_copy(
                    rows_v.at[pl.ds(0, rem)],
                    acc_sh.at[pl.ds(base + (RPT_LAST // CH) * CH, rem)])

        plsc.subcore_barrier()

        wait_i(sidx_a, isas)
        pltpu.async_copy(x_hbm.at[sidx_a], rows_a, gsa)

        def pair(i, carry):
            c = 2 * i
            wait_rows(rows_a, gsa)
            load_sidx(c + 2, sidx_a, isas)
            wait_i(didx_a, isad)
            sa = pltpu.async_copy(rows_a, acc_sh.at[didx_a], ssa, add=True)
            wait_i(sidx_b, isbs)
            gb = pltpu.async_copy(x_hbm.at[sidx_b], rows_b, gsb)
            sa.wait()
            load_didx(c + 2, didx_a, isad)
            wait_i(sidx_a, isas)
            gb.wait()
            pltpu.async_copy(x_hbm.at[sidx_a], rows_a, gsa)
            wait_i(didx_b, isbd)
            sb = pltpu.async_copy(rows_b, acc_sh.at[didx_b], ssb, add=True)

            @pl.when(c + 3 < NCH)
            def _():
                load_sidx(c + 3, sidx_b, isbs)

            sb.wait()

            @pl.when(c + 3 < NCH)
            def _():
                load_didx(c + 3, didx_b, isbd)

            return carry

        lax.fori_loop(0, NCH // 2, pair, 0)
        wait_rows(rows_a, gsa)
        wait_i(didx_a, isad)
        pltpu.sync_copy(rows_a, acc_sh.at[didx_a], add=True)
        plsc.subcore_barrier()

        @pl.when(sid < _NS - 1)
        def _out_main():
            pltpu.sync_copy(acc_sh.at[pl.ds(sid * RPT, RPT)],
                            out_hbm.at[cid, pl.ds(sid * RPT, RPT)])

        @pl.when(sid == _NS - 1)
        def _out_last():
            base = (_NS - 1) * RPT
            pltpu.sync_copy(acc_sh.at[pl.ds(base, RPT_LAST)],
                            out_hbm.at[cid, pl.ds(base, RPT_LAST)])

    return agg_kernel(x, src, dst)


def _tc_mlp(x, agg, W1, b1, W2, b2):
    N, D = x.shape
    H = W1.shape[1]
    BR = 2000

    def body(x_ref, a_ref, w1_ref, b1_ref, w2_ref, b2_ref, o_ref):
        s = x_ref[...] + a_ref[0] + a_ref[1]
        h = jnp.dot(s, w1_ref[...], preferred_element_type=jnp.float32)
        h = jnp.maximum(h + b1_ref[...], 0.0)
        o_ref[...] = (jnp.dot(h, w2_ref[...],
                              preferred_element_type=jnp.float32) + b2_ref[...])

    return pl.pallas_call(
        body,
        grid=(N // BR,),
        in_specs=[
            pl.BlockSpec((BR, D), lambda i: (i, 0)),
            pl.BlockSpec((_NC, BR, D), lambda i: (0, i, 0)),
            pl.BlockSpec((D, H), lambda i: (0, 0)),
            pl.BlockSpec((1, H), lambda i: (0, 0)),
            pl.BlockSpec((H, H), lambda i: (0, 0)),
            pl.BlockSpec((1, H), lambda i: (0, 0)),
        ],
        out_specs=pl.BlockSpec((BR, H), lambda i: (i, 0)),
        out_shape=jax.ShapeDtypeStruct((N, H), jnp.float32),
    )(x, agg, W1, b1, W2, b2)


def _sc_gather(x, uidx, vidx):
    N, D = x.shape
    B = uidx.shape[0]
    BW = B // _NW
    mesh = plsc.VectorSubcoreMesh(core_axis_name="c", subcore_axis_name="s")

    @functools.partial(
        pl.kernel,
        out_type=(jax.ShapeDtypeStruct((B, D), jnp.float32),
                  jax.ShapeDtypeStruct((B, D), jnp.float32)),
        mesh=mesh,
        scratch_types=[
            pltpu.VMEM((BW,), jnp.int32),
            pltpu.VMEM((BW, D), jnp.float32),
            pltpu.SemaphoreType.DMA,
        ],
    )
    def gather_kernel(x_hbm, u_hbm, v_hbm, ou_hbm, ov_hbm, idx_v, rows_v, sem):
        cid = lax.axis_index("c")
        sid = lax.axis_index("s")
        base = (cid * _NS + sid) * BW
        pltpu.sync_copy(u_hbm.at[pl.ds(base, BW)], idx_v)
        pltpu.async_copy(x_hbm.at[idx_v], rows_v, sem).wait()
        pltpu.sync_copy(rows_v, ou_hbm.at[pl.ds(base, BW)])
        pltpu.sync_copy(v_hbm.at[pl.ds(base, BW)], idx_v)
        pltpu.async_copy(x_hbm.at[idx_v], rows_v, sem).wait()
        pltpu.sync_copy(rows_v, ov_hbm.at[pl.ds(base, BW)])

    return gather_kernel(x, uidx, vidx)


def _tc_score(xu, xv, fu, fv, tb, ep_W1, ep_b1, ep_W2, ep_b2,
              te_W1, te_b1, te_W2, te_b2):
    Cn, H = xu.shape

    def body(xu_ref, xv_ref, fu_ref, fv_ref, tb_ref, w1_ref, b1_ref, w2_ref,
             b2_ref, tw1_ref, tb1_ref, tw2_ref, tb2_ref, o_ref):
        w1 = w1_ref[...]
        s = xu_ref[...] + xv_ref[...]
        d = jnp.abs(xu_ref[...] - xv_ref[...])
        ffs = fu_ref[...] + fv_ref[...]
        ffd = jnp.abs(fu_ref[...] - fv_ref[...])
        te = jnp.maximum(tb_ref[...] * tw1_ref[...] + tb1_ref[...], 0.0)
        temb = (jnp.dot(te, tw2_ref[...], preferred_element_type=jnp.float32)
                + tb2_ref[...])
        cvec = (jnp.dot(ffs, w1[0:H, :], preferred_element_type=jnp.float32)
                + jnp.dot(ffd, w1[H:2 * H, :],
                          preferred_element_type=jnp.float32)
                + jnp.dot(temb, w1[4 * H:5 * H, :],
                          preferred_element_type=jnp.float32)
                + b1_ref[...])
        pre = (jnp.dot(s, w1[2 * H:3 * H, :],
                       preferred_element_type=jnp.float32)
               + jnp.dot(d, w1[3 * H:4 * H, :],
                         preferred_element_type=jnp.float32)
               + cvec)
        h = jnp.maximum(pre, 0.0)
        o_ref[...] = (jnp.dot(h, w2_ref[...],
                              preferred_element_type=jnp.float32) + b2_ref[...])

    return pl.pallas_call(
        body,
        out_shape=jax.ShapeDtypeStruct((Cn, 1), jnp.float32),
    )(xu, xv, fu, fv, tb, ep_W1, ep_b1, ep_W2, ep_b2,
      te_W1, te_b1, te_W2, te_b2)


def kernel(x, edge_index, first_edge, candidate_edges, t,
           gin0_W1, gin0_b1, gin0_W2, gin0_b2,
           gin1_W1, gin1_b1, gin1_W2, gin1_b2,
           gin2_W1, gin2_b1, gin2_W2, gin2_b2,
           ep_W1, ep_b1, ep_W2, ep_b2,
           te_W1, te_b1, te_W2, te_b2):
    N, D = x.shape
    H = gin0_W1.shape[1]
    src = edge_index[0]
    dst = edge_index[1]

    layers = ((gin0_W1, gin0_b1, gin0_W2, gin0_b2),
              (gin1_W1, gin1_b1, gin1_W2, gin1_b2),
              (gin2_W1, gin2_b1, gin2_W2, gin2_b2))
    for W1, b1, W2, b2 in layers:
        agg = _sc_aggregate(x, src, dst)
        x = _tc_mlp(x, agg, W1, b1.reshape(1, H), W2, b2.reshape(1, H))

    Cn = candidate_edges.shape[0]
    B = ((Cn + 1 + 8 * _NW - 1) // (8 * _NW)) * (8 * _NW)
    pad = jnp.arange(B - Cn - 1, dtype=jnp.int32) % N
    uidx = jnp.concatenate([candidate_edges[:, 0].astype(jnp.int32),
                            first_edge[0:1].astype(jnp.int32), pad])
    vidx = jnp.concatenate([candidate_edges[:, 1].astype(jnp.int32),
                            first_edge[1:2].astype(jnp.int32), pad])
    xu_all, xv_all = _sc_gather(x, uidx, vidx)
    xu = xu_all[:Cn]
    xv = xv_all[:Cn]
    fu = xu_all[Cn:Cn + 1]
    fv = xv_all[Cn:Cn + 1]

    tb = jnp.full((1, H), t, dtype=jnp.float32)
    scores = _tc_score(xu, xv, fu, fv, tb,
                       ep_W1, ep_b1.reshape(1, H), ep_W2,
                       ep_b2.reshape(1, 1),
                       te_W1, te_b1.reshape(1, H), te_W2,
                       te_b2.reshape(1, H))
    return scores.reshape(-1)


# speedup vs baseline: 9.4827x; 1.1151x over previous
import functools

import jax
import jax.numpy as jnp
from jax import lax
from jax.experimental import pallas as pl
from jax.experimental.pallas import tpu as pltpu
from jax.experimental.pallas import tpu_sc as plsc

_NC = 2
_NS = 16
_NW = _NC * _NS


def _sc_aggregate(x, src, dst):
    N, D = x.shape
    E = src.shape[0]
    EW = E // _NW
    CH = 80
    NCH = EW // CH
    RPT = (N // _NS) // 8 * 8
    RPT_LAST = N - (_NS - 1) * RPT
    mesh = plsc.VectorSubcoreMesh(core_axis_name="c", subcore_axis_name="s")

    @functools.partial(
        pl.kernel,
        out_type=jax.ShapeDtypeStruct((_NC, N, D), jnp.float32),
        mesh=mesh,
        scratch_types=(
            [pltpu.VMEM((CH,), jnp.int32)] * 8
            + [pltpu.VMEM((CH, D), jnp.float32)] * 4
            + [pltpu.VMEM_SHARED((N, D), jnp.float32)]
            + [pltpu.SemaphoreType.DMA] * 12
        ),
    )
    def agg_kernel(x_hbm, src_hbm, dst_hbm, out_hbm,
                   si0, si1, si2, si3, di0, di1, di2, di3,
                   r0, r1, r2, r3, acc_sh, *sems):
        cid = lax.axis_index("c")
        sid = lax.axis_index("s")
        wid = cid * _NS + sid
        sidx = (si0, si1, si2, si3)
        didx = (di0, di1, di2, di3)
        rows = (r0, r1, r2, r3)
        isem = sems[0:4]
        gsem = sems[4:8]
        ssem = sems[8:12]
        rows_v = r0
        ebase = wid * EW

        def load_idx(c, k):
            pltpu.async_copy(src_hbm.at[pl.ds(ebase + c * CH, CH)], sidx[k],
                             isem[k])
            pltpu.async_copy(dst_hbm.at[pl.ds(ebase + c * CH, CH)], didx[k],
                             isem[k])

        def wait_idx(k):
            pltpu.make_async_copy(src_hbm.at[pl.ds(0, CH)], sidx[k],
                                  isem[k]).wait()
            pltpu.make_async_copy(dst_hbm.at[pl.ds(0, CH)], didx[k],
                                  isem[k]).wait()

        def wait_rows(k):
            pltpu.make_async_copy(x_hbm.at[pl.ds(0, CH)], rows[k],
                                  gsem[k]).wait()

        for k in range(4):
            load_idx(k, k)

        zeros16 = jnp.zeros((16,), jnp.float32)

        def zrow(i, carry):
            for j in range(D // 16):
                rows_v[i, pl.ds(j * 16, 16)] = zeros16
            return carry

        lax.fori_loop(0, CH, zrow, 0)

        @pl.when(sid < _NS - 1)
        def _zero_main():
            for k in range(RPT // CH):
                pltpu.sync_copy(rows_v,
                                acc_sh.at[pl.ds(sid * RPT + k * CH, CH)])
            rem = RPT % CH
            if rem:
                pltpu.sync_copy(
                    rows_v.at[pl.ds(0, rem)],
                    acc_sh.at[pl.ds(sid * RPT + (RPT // CH) * CH, rem)])

        @pl.when(sid == _NS - 1)
        def _zero_last():
            base = (_NS - 1) * RPT
            for k in range(RPT_LAST // CH):
                pltpu.sync_copy(rows_v, acc_sh.at[pl.ds(base + k * CH, CH)])
            rem = RPT_LAST % CH
            if rem:
                pltpu.sync_copy(
                    rows_v.at[pl.ds(0, rem)],
                    acc_sh.at[pl.ds(base + (RPT_LAST // CH) * CH, rem)])

        plsc.subcore_barrier()

        for k in range(4):
            wait_idx(k)
            pltpu.async_copy(x_hbm.at[sidx[k]], rows[k], gsem[k])

        def group(i, carry):
            c = 4 * i
            scat = []
            for k in range(4):
                wait_rows(k)
                scat.append(pltpu.async_copy(
                    rows[k], acc_sh.at[didx[k]], ssem[k], add=True))
            for k in range(4):
                scat[k].wait()

                @pl.when(c + k + 4 < NCH)
                def _(k=k):
                    load_idx(c + k + 4, k)

            for k in range(4):
                @pl.when(c + k + 4 < NCH)
                def _(k=k):
                    wait_idx(k)
                    pltpu.async_copy(x_hbm.at[sidx[k]], rows[k], gsem[k])

            return carry

        lax.fori_loop(0, NCH // 4, group, 0)
        wait_rows(0)
        pltpu.sync_copy(rows[0], acc_sh.at[didx[0]], add=True)
        plsc.subcore_barrier()

        @pl.when(sid < _NS - 1)
        def _out_main():
            pltpu.sync_copy(acc_sh.at[pl.ds(sid * RPT, RPT)],
                            out_hbm.at[cid, pl.ds(sid * RPT, RPT)])

        @pl.when(sid == _NS - 1)
        def _out_last():
            base = (_NS - 1) * RPT
            pltpu.sync_copy(acc_sh.at[pl.ds(base, RPT_LAST)],
                            out_hbm.at[cid, pl.ds(base, RPT_LAST)])

    return agg_kernel(x, src, dst)


def _tc_mlp(x, agg, W1, b1, W2, b2):
    N, D = x.shape
    H = W1.shape[1]
    BR = 2000

    def body(x_ref, a_ref, w1_ref, b1_ref, w2_ref, b2_ref, o_ref):
        s = x_ref[...] + a_ref[0] + a_ref[1]
        h = jnp.dot(s, w1_ref[...], preferred_element_type=jnp.float32)
        h = jnp.maximum(h + b1_ref[...], 0.0)
        o_ref[...] = (jnp.dot(h, w2_ref[...],
                              preferred_element_type=jnp.float32) + b2_ref[...])

    return pl.pallas_call(
        body,
        grid=(N // BR,),
        in_specs=[
            pl.BlockSpec((BR, D), lambda i: (i, 0)),
            pl.BlockSpec((_NC, BR, D), lambda i: (0, i, 0)),
            pl.BlockSpec((D, H), lambda i: (0, 0)),
            pl.BlockSpec((1, H), lambda i: (0, 0)),
            pl.BlockSpec((H, H), lambda i: (0, 0)),
            pl.BlockSpec((1, H), lambda i: (0, 0)),
        ],
        out_specs=pl.BlockSpec((BR, H), lambda i: (i, 0)),
        out_shape=jax.ShapeDtypeStruct((N, H), jnp.float32),
    )(x, agg, W1, b1, W2, b2)


def _sc_gather(x, uidx, vidx):
    N, D = x.shape
    B = uidx.shape[0]
    BW = B // _NW
    mesh = plsc.VectorSubcoreMesh(core_axis_name="c", subcore_axis_name="s")

    @functools.partial(
        pl.kernel,
        out_type=(jax.ShapeDtypeStruct((B, D), jnp.float32),
                  jax.ShapeDtypeStruct((B, D), jnp.float32)),
        mesh=mesh,
        scratch_types=[
            pltpu.VMEM((BW,), jnp.int32),
            pltpu.VMEM((BW, D), jnp.float32),
            pltpu.SemaphoreType.DMA,
        ],
    )
    def gather_kernel(x_hbm, u_hbm, v_hbm, ou_hbm, ov_hbm, idx_v, rows_v, sem):
        cid = lax.axis_index("c")
        sid = lax.axis_index("s")
        base = (cid * _NS + sid) * BW
        pltpu.sync_copy(u_hbm.at[pl.ds(base, BW)], idx_v)
        pltpu.async_copy(x_hbm.at[idx_v], rows_v, sem).wait()
        pltpu.sync_copy(rows_v, ou_hbm.at[pl.ds(base, BW)])
        pltpu.sync_copy(v_hbm.at[pl.ds(base, BW)], idx_v)
        pltpu.async_copy(x_hbm.at[idx_v], rows_v, sem).wait()
        pltpu.sync_copy(rows_v, ov_hbm.at[pl.ds(base, BW)])

    return gather_kernel(x, uidx, vidx)


def _tc_score(xu, xv, fu, fv, tb, ep_W1, ep_b1, ep_W2, ep_b2,
              te_W1, te_b1, te_W2, te_b2):
    Cn, H = xu.shape

    def body(xu_ref, xv_ref, fu_ref, fv_ref, tb_ref, w1_ref, b1_ref, w2_ref,
             b2_ref, tw1_ref, tb1_ref, tw2_ref, tb2_ref, o_ref):
        w1 = w1_ref[...]
        s = xu_ref[...] + xv_ref[...]
        d = jnp.abs(xu_ref[...] - xv_ref[...])
        ffs = fu_ref[...] + fv_ref[...]
        ffd = jnp.abs(fu_ref[...] - fv_ref[...])
        te = jnp.maximum(tb_ref[...] * tw1_ref[...] + tb1_ref[...], 0.0)
        temb = (jnp.dot(te, tw2_ref[...], preferred_element_type=jnp.float32)
                + tb2_ref[...])
        cvec = (jnp.dot(ffs, w1[0:H, :], preferred_element_type=jnp.float32)
                + jnp.dot(ffd, w1[H:2 * H, :],
                          preferred_element_type=jnp.float32)
                + jnp.dot(temb, w1[4 * H:5 * H, :],
                          preferred_element_type=jnp.float32)
                + b1_ref[...])
        pre = (jnp.dot(s, w1[2 * H:3 * H, :],
                       preferred_element_type=jnp.float32)
               + jnp.dot(d, w1[3 * H:4 * H, :],
                         preferred_element_type=jnp.float32)
               + cvec)
        h = jnp.maximum(pre, 0.0)
        o_ref[...] = (jnp.dot(h, w2_ref[...],
                              preferred_element_type=jnp.float32) + b2_ref[...])

    return pl.pallas_call(
        body,
        out_shape=jax.ShapeDtypeStruct((Cn, 1), jnp.float32),
    )(xu, xv, fu, fv, tb, ep_W1, ep_b1, ep_W2, ep_b2,
      te_W1, te_b1, te_W2, te_b2)


def kernel(x, edge_index, first_edge, candidate_edges, t,
           gin0_W1, gin0_b1, gin0_W2, gin0_b2,
           gin1_W1, gin1_b1, gin1_W2, gin1_b2,
           gin2_W1, gin2_b1, gin2_W2, gin2_b2,
           ep_W1, ep_b1, ep_W2, ep_b2,
           te_W1, te_b1, te_W2, te_b2):
    N, D = x.shape
    H = gin0_W1.shape[1]
    src = edge_index[0]
    dst = edge_index[1]

    layers = ((gin0_W1, gin0_b1, gin0_W2, gin0_b2),
              (gin1_W1, gin1_b1, gin1_W2, gin1_b2),
              (gin2_W1, gin2_b1, gin2_W2, gin2_b2))
    for W1, b1, W2, b2 in layers:
        agg = _sc_aggregate(x, src, dst)
        x = _tc_mlp(x, agg, W1, b1.reshape(1, H), W2, b2.reshape(1, H))

    Cn = candidate_edges.shape[0]
    B = ((Cn + 1 + 8 * _NW - 1) // (8 * _NW)) * (8 * _NW)
    pad = jnp.arange(B - Cn - 1, dtype=jnp.int32) % N
    uidx = jnp.concatenate([candidate_edges[:, 0].astype(jnp.int32),
                            first_edge[0:1].astype(jnp.int32), pad])
    vidx = jnp.concatenate([candidate_edges[:, 1].astype(jnp.int32),
                            first_edge[1:2].astype(jnp.int32), pad])
    xu_all, xv_all = _sc_gather(x, uidx, vidx)
    xu = xu_all[:Cn]
    xv = xv_all[:Cn]
    fu = xu_all[Cn:Cn + 1]
    fv = xv_all[Cn:Cn + 1]

    tb = jnp.full((1, H), t, dtype=jnp.float32)
    scores = _tc_score(xu, xv, fu, fv, tb,
                       ep_W1, ep_b1.reshape(1, H), ep_W2,
                       ep_b2.reshape(1, 1),
                       te_W1, te_b1.reshape(1, H), te_W2,
                       te_b2.reshape(1, H))
    return scores.reshape(-1)
